# baseline (speedup 1.0000x reference)
import functools
import math

import jax
import jax.numpy as jnp
import numpy as np
from jax import lax
from jax.experimental import pallas as pl
from jax.experimental.pallas import tpu as pltpu
from jax.experimental.pallas import tpu_sc as plsc

_DIMS = [1, 1, 2, 2, 2, 2, 2]
_NUM_HEADS = 8
_NUM_LAYERS = 4
_MULT = 64
_MULT_FF = 256
_CODEBOOK = 8192


def _build_Q():
    j = np.arange(12)
    qs = [np.ones((1, 12)) / np.sqrt(12.0)]
    qs.append((((-1.0) ** j).reshape(1, 12)) / np.sqrt(12.0))
    for m in range(1, 6):
        c = np.cos(2.0 * np.pi * m * j / 12.0)
        s = np.sin(2.0 * np.pi * m * j / 12.0)
        qs.append(np.stack([c / np.linalg.norm(c), s / np.linalg.norm(s)], axis=0))
    return [jnp.asarray(q, dtype=jnp.float32) for q in qs]


_Q = _build_Q()


def _lin(vs, W):
    return [vs[i] @ W[i] for i in range(len(vs))]


def _act(vs):
    out = []
    for i in range(len(vs)):
        Q = _Q[i]
        p = jax.nn.gelu(jnp.einsum('db,...dm->...bm', Q, vs[i]), approximate=False)
        out.append(jnp.einsum('db,...bm->...dm', Q, p))
    return out


def _featurize(x, bias, weight):
    xv = (x + bias)[..., None]
    return [jnp.einsum('db,...bo->...do', Q, xv) * weight for Q in _Q]


def _pos_encoding(vs):
    S = vs[0].shape[1]
    position = jnp.arange(S, dtype=jnp.float32)[:, None]
    div = jnp.exp(jnp.arange(0, _MULT, 2, dtype=jnp.float32) * (-(math.log(10000.0) / _MULT)))
    pe = jnp.zeros((S, _MULT), dtype=jnp.float32)
    pe = pe.at[:, 0::2].set(jnp.sin(position * div))
    pe = pe.at[:, 1::2].set(jnp.cos(position * div))
    out = []
    for i in range(len(vs)):
        qsum = _Q[i].sum(axis=1)
        enc = qsum[None, :, None] * pe[:, None, :]
        out.append(vs[i] + enc[None])
    return out


def _split_heads(xs):
    hs = []
    for x in xs:
        B, S, d, m = x.shape
        mk = m // _NUM_HEADS
        hs.append(x.reshape(B, S, d, _NUM_HEADS, mk).transpose(0, 3, 1, 2, 4).reshape(B, _NUM_HEADS, S, d * mk))
    return hs


def _mha(vs, Wq, Wk, Wv, Wo):
    qh = _split_heads(_lin(vs, Wq))
    kh = _split_heads(_lin(vs, Wk))
    vh = _split_heads(_lin(vs, Wv))
    Qc = jnp.concatenate(qh, axis=-1)
    Kc = jnp.concatenate(kh, axis=-1)
    scores = jnp.matmul(Qc, jnp.swapaxes(Kc, -2, -1)) / math.sqrt(Qc.shape[-1])
    probs = jax.nn.softmax(scores, axis=-1)
    outs = []
    for i in range(len(vh)):
        o = jnp.matmul(probs, vh[i])
        B, H, S, dm = o.shape
        d = _DIMS[i]
        mk = dm // d
        outs.append(o.reshape(B, H, S, d, mk).transpose(0, 2, 3, 1, 4).reshape(B, S, d, H * mk))
    return _lin(outs, Wo)


def _encoder_layer(vs, Wq, Wk, Wv, Wo, F1, F2):
    a = _mha(vs, Wq, Wk, Wv, Wo)
    vs = [vs[i] + a[i] for i in range(len(vs))]
    f = _lin(_act(_lin(vs, F1)), F2)
    return [vs[i] + f[i] for i in range(len(vs))]


def _fc_out(vs, out_W):
    vs = _lin(vs, out_W)
    parts = [jnp.einsum('db,...dm->...bm', _Q[i], vs[i]) for i in range(len(vs))]
    perm = jnp.concatenate(parts, axis=-1)
    return jnp.mean(perm, axis=-1)


def _fold_sum_lanes(x, width):
    p = 1
    while p < width:
        p *= 2
    if p != width:
        x = jnp.concatenate(
            [x, jnp.zeros(x.shape[:-1] + (p - width,), x.dtype)], axis=-1)
    while p > 1:
        h = p // 2
        x = x[..., :h] + x[..., h:p]
        p = h
    return x


def _vq_body(flat_ref, cb_ref, cb2t_ref, idx_ref):
    flat = flat_ref[...]
    cb = cb_ref[...]
    ff = flat * flat
    a = _fold_sum_lanes(ff, 12)
    mm = lax.dot_general(flat, cb, (((1,), (1,)), ((), ())),
                         preferred_element_type=jnp.float32)
    d2 = (a - 2.0 * mm) + cb2t_ref[...]
    idx_ref[...] = jnp.argmin(d2, axis=1).astype(jnp.int32)[:, None]


def _vq_argmin(flat, cb, cb2t):
    n = flat.shape[0]
    return pl.pallas_call(
        _vq_body,
        out_shape=jax.ShapeDtypeStruct((n, 1), jnp.int32),
    )(flat, cb, cb2t)


def _cb2_body(cb_ref, out_ref):
    cb = cb_ref[...]
    cc = cb * cb
    out_ref[...] = jnp.transpose(_fold_sum_lanes(cc, 12))


def _cb2t(cb):
    return pl.pallas_call(
        _cb2_body,
        out_shape=jax.ShapeDtypeStruct((1, cb.shape[0]), jnp.float32),
    )(cb)


def _make_sc_gather(B, D):
    info = plsc.get_sparse_core_info()
    NC, NS = info.num_cores, info.num_subcores
    NW = NC * NS
    b_per_w = B // NW
    mesh = plsc.VectorSubcoreMesh(core_axis_name="c", subcore_axis_name="s")

    @functools.partial(
        pl.kernel, mesh=mesh,
        out_type=jax.ShapeDtypeStruct((B, D), jnp.float32),
        scratch_types=[
            pltpu.VMEM((b_per_w,), jnp.int32),
            pltpu.VMEM((b_per_w, D), jnp.float32),
            pltpu.SemaphoreType.DMA,
        ],
    )
    def k(table_hbm, idx_hbm, out_hbm, idx_v, rows_v, sem):
        wid = lax.axis_index("s") * NC + lax.axis_index("c")
        base = wid * b_per_w
        pltpu.sync_copy(idx_hbm.at[pl.ds(base, b_per_w)], idx_v)
        pltpu.async_copy(table_hbm.at[idx_v], rows_v, sem).wait()
        pltpu.sync_copy(rows_v, out_hbm.at[pl.ds(base, b_per_w)])

    return k


def kernel(x, feat_bias, feat_weight, emb_W1, emb_W2, emb_W3,
           attn_Wq, attn_Wk, attn_Wv, attn_Wo, ff_W1, ff_W2, out_W, codebook):
    vs = _featurize(x, feat_bias, feat_weight)
    vs = _lin(vs, emb_W1)
    vs = _act(vs)
    vs = _lin(vs, emb_W2)
    vs = _act(vs)
    vs = _lin(vs, emb_W3)
    vs = _pos_encoding(vs)
    for l in range(_NUM_LAYERS):
        vs = _encoder_layer(vs, attn_Wq[l], attn_Wk[l], attn_Wv[l],
                            attn_Wo[l], ff_W1[l], ff_W2[l])
    z_e = _fc_out(vs, out_W)

    B, S, D = z_e.shape
    flat = z_e.reshape(-1, D)
    cb2t = _cb2t(codebook)
    idx = _vq_argmin(flat, codebook, cb2t)[:, 0]

    cb_pad = jnp.concatenate(
        [codebook, jnp.zeros((codebook.shape[0], 116), jnp.float32)], axis=1)
    gathered = _make_sc_gather(B * S, 128)(cb_pad, idx)
    z_q = gathered[:, :D].reshape(B, S, D)

    z_q_st = z_e + lax.stop_gradient(z_q - z_e)
    return z_q_st, idx.reshape(B, S)


# speedup vs baseline: 1.0694x; 1.0694x over previous
import functools
import math

import jax
import jax.numpy as jnp
import numpy as np
from jax import lax
from jax.experimental import pallas as pl
from jax.experimental.pallas import tpu as pltpu
from jax.experimental.pallas import tpu_sc as plsc

_DIMS = [1, 1, 2, 2, 2, 2, 2]
_NUM_HEADS = 8
_NUM_LAYERS = 4
_MULT = 64
_MULT_FF = 256
_CODEBOOK = 8192


def _build_Q():
    j = np.arange(12)
    qs = [np.ones((1, 12)) / np.sqrt(12.0)]
    qs.append((((-1.0) ** j).reshape(1, 12)) / np.sqrt(12.0))
    for m in range(1, 6):
        c = np.cos(2.0 * np.pi * m * j / 12.0)
        s = np.sin(2.0 * np.pi * m * j / 12.0)
        qs.append(np.stack([c / np.linalg.norm(c), s / np.linalg.norm(s)], axis=0))
    return [jnp.asarray(q, dtype=jnp.float32) for q in qs]


_Q = _build_Q()


def _lin(vs, W):
    return [vs[i] @ W[i] for i in range(len(vs))]


def _act(vs):
    out = []
    for i in range(len(vs)):
        Q = _Q[i]
        p = jax.nn.gelu(jnp.einsum('db,...dm->...bm', Q, vs[i]), approximate=False)
        out.append(jnp.einsum('db,...bm->...dm', Q, p))
    return out


def _featurize(x, bias, weight):
    xv = (x + bias)[..., None]
    return [jnp.einsum('db,...bo->...do', Q, xv) * weight for Q in _Q]


def _pos_encoding(vs):
    S = vs[0].shape[1]
    position = jnp.arange(S, dtype=jnp.float32)[:, None]
    div = jnp.exp(jnp.arange(0, _MULT, 2, dtype=jnp.float32) * (-(math.log(10000.0) / _MULT)))
    pe = jnp.zeros((S, _MULT), dtype=jnp.float32)
    pe = pe.at[:, 0::2].set(jnp.sin(position * div))
    pe = pe.at[:, 1::2].set(jnp.cos(position * div))
    out = []
    for i in range(len(vs)):
        qsum = _Q[i].sum(axis=1)
        enc = qsum[None, :, None] * pe[:, None, :]
        out.append(vs[i] + enc[None])
    return out


_OFFS = [0, 1, 2, 4, 6, 8, 10]
_S = 512


def _attnA_body(*refs):
    v_refs = refs[0:7]
    wq_ref, wk_ref, wv_ref = refs[7:10]
    sc_ref = refs[10]
    vp_refs = refs[11:18]
    qs, ks = [], []
    for i, d in enumerate(_DIMS):
        v_i = v_refs[i][0].reshape(_S * d, _MULT)
        q_i = jnp.dot(v_i, wq_ref[i], preferred_element_type=jnp.float32)
        k_i = jnp.dot(v_i, wk_ref[i], preferred_element_type=jnp.float32)
        p_i = jnp.dot(v_i, wv_ref[i], preferred_element_type=jnp.float32)
        qs.append(q_i.reshape(_S, d, _MULT))
        ks.append(k_i.reshape(_S, d, _MULT))
        vp_refs[i][0] = p_i.reshape(1, _S, d, _MULT)[0]
    for h in range(_NUM_HEADS):
        qc = jnp.concatenate(
            [q[:, j, 8 * h:8 * h + 8] for q, d in zip(qs, _DIMS) for j in range(d)], axis=1)
        kc = jnp.concatenate(
            [k[:, j, 8 * h:8 * h + 8] for k, d in zip(ks, _DIMS) for j in range(d)], axis=1)
        sc_ref[0, h] = lax.dot_general(
            qc, kc, (((1,), (1,)), ((), ())), preferred_element_type=jnp.float32)


def _attnA(vs, Wq, Wk, Wv):
    B = vs[0].shape[0]
    w_spec = pl.BlockSpec((7, _MULT, _MULT), lambda b: (0, 0, 0))
    v_specs = [pl.BlockSpec((1, _S, d, _MULT), lambda b: (b, 0, 0, 0)) for d in _DIMS]
    outs = pl.pallas_call(
        _attnA_body,
        grid=(B,),
        in_specs=v_specs + [w_spec, w_spec, w_spec],
        out_specs=[pl.BlockSpec((1, _NUM_HEADS, _S, _S), lambda b: (b, 0, 0, 0))] + v_specs,
        out_shape=[jax.ShapeDtypeStruct((B, _NUM_HEADS, _S, _S), jnp.float32)] +
                  [jax.ShapeDtypeStruct((B, _S, d, _MULT), jnp.float32) for d in _DIMS],
    )(*vs, Wq, Wk, Wv)
    return outs[0], list(outs[1:])


def _split_heads(xs):
    hs = []
    for x in xs:
        B, S, d, m = x.shape
        mk = m // _NUM_HEADS
        hs.append(x.reshape(B, S, d, _NUM_HEADS, mk).transpose(0, 3, 1, 2, 4).reshape(B, _NUM_HEADS, S, d * mk))
    return hs


def _attnB_body(*refs):
    o_ref = refs[0]
    vs_refs = refs[1:8]
    wo_ref, f1_ref = refs[8], refs[9]
    f_refs = refs[10:17]
    v2_refs = refs[17:24]
    for i, d in enumerate(_DIMS):
        off = _OFFS[i]
        cols = []
        for h in range(_NUM_HEADS):
            cols.append(o_ref[0, h][:, 8 * off:8 * (off + d)])
        js = []
        for j in range(d):
            js.append(jnp.concatenate(
                [cols[h][:, 8 * j:8 * j + 8] for h in range(_NUM_HEADS)],
                axis=1)[:, None, :])
        attn_rows = jnp.concatenate(js, axis=1).reshape(_S * d, _MULT)
        a_i = jnp.dot(attn_rows, wo_ref[i], preferred_element_type=jnp.float32)
        v2_i = vs_refs[i][0].reshape(_S * d, _MULT) + a_i
        f_i = jnp.dot(v2_i, f1_ref[i], preferred_element_type=jnp.float32)
        v2_refs[i][0] = v2_i.reshape(_S, d, _MULT)
        f_refs[i][0] = f_i.reshape(_S, d, _MULT_FF)


def _attnB(o_cat, vs, Wo, F1):
    B = o_cat.shape[0]
    v_specs = [pl.BlockSpec((1, _S, d, _MULT), lambda b: (b, 0, 0, 0)) for d in _DIMS]
    f_specs = [pl.BlockSpec((1, _S, d, _MULT_FF), lambda b: (b, 0, 0, 0)) for d in _DIMS]
    outs = pl.pallas_call(
        _attnB_body,
        grid=(B,),
        in_specs=[pl.BlockSpec((1, _NUM_HEADS, _S, 96), lambda b: (b, 0, 0, 0))]
                 + v_specs
                 + [pl.BlockSpec((7, _MULT, _MULT), lambda b: (0, 0, 0)),
                    pl.BlockSpec((7, _MULT, _MULT_FF), lambda b: (0, 0, 0))],
        out_specs=f_specs + v_specs,
        out_shape=[jax.ShapeDtypeStruct((B, _S, d, _MULT_FF), jnp.float32) for d in _DIMS]
                  + [jax.ShapeDtypeStruct((B, _S, d, _MULT), jnp.float32) for d in _DIMS],
    )(o_cat, *vs, Wo, F1)
    return list(outs[:7]), list(outs[7:])


def _ffC_body(*refs):
    g_refs = refs[0:7]
    v2_refs = refs[7:14]
    f2_ref = refs[14]
    o_refs = refs[15:22]
    for i, d in enumerate(_DIMS):
        g_i = g_refs[i][0].reshape(_S * d, _MULT_FF)
        f_i = jnp.dot(g_i, f2_ref[i], preferred_element_type=jnp.float32)
        o_refs[i][0] = (v2_refs[i][0].reshape(_S * d, _MULT) + f_i).reshape(_S, d, _MULT)


def _ffC(g, v2, F2):
    B = g[0].shape[0]
    v_specs = [pl.BlockSpec((1, _S, d, _MULT), lambda b: (b, 0, 0, 0)) for d in _DIMS]
    f_specs = [pl.BlockSpec((1, _S, d, _MULT_FF), lambda b: (b, 0, 0, 0)) for d in _DIMS]
    outs = pl.pallas_call(
        _ffC_body,
        grid=(B,),
        in_specs=f_specs + v_specs
                 + [pl.BlockSpec((7, _MULT_FF, _MULT), lambda b: (0, 0, 0))],
        out_specs=v_specs,
        out_shape=[jax.ShapeDtypeStruct((B, _S, d, _MULT), jnp.float32) for d in _DIMS],
    )(*g, *v2, F2)
    return list(outs)


def _encoder_layer(vs, Wq, Wk, Wv, Wo, F1, F2):
    scores_raw, vp = _attnA(vs, Wq, Wk, Wv)
    probs = jax.nn.softmax(scores_raw / math.sqrt(96), axis=-1)
    vh_cat = jnp.concatenate(_split_heads(vp), axis=-1)
    o_cat = jnp.matmul(probs, vh_cat)
    f, v2 = _attnB(o_cat, vs, Wo, F1)
    g = _act(f)
    return _ffC(g, v2, F2)


def _fc_out(vs, out_W):
    vs = _lin(vs, out_W)
    parts = [jnp.einsum('db,...dm->...bm', _Q[i], vs[i]) for i in range(len(vs))]
    perm = jnp.concatenate(parts, axis=-1)
    return jnp.mean(perm, axis=-1)


def _fold_sum_lanes(x, width):
    p = 1
    while p < width:
        p *= 2
    if p != width:
        x = jnp.concatenate(
            [x, jnp.zeros(x.shape[:-1] + (p - width,), x.dtype)], axis=-1)
    while p > 1:
        h = p // 2
        x = x[..., :h] + x[..., h:p]
        p = h
    return x


def _vq_body(flat_ref, cb_ref, cb2t_ref, idx_ref):
    flat = flat_ref[...]
    cb = cb_ref[...]
    ff = flat * flat
    a = _fold_sum_lanes(ff, 12)
    mm = lax.dot_general(flat, cb, (((1,), (1,)), ((), ())),
                         preferred_element_type=jnp.float32)
    d2 = (a - 2.0 * mm) + cb2t_ref[...]
    idx_ref[...] = jnp.argmin(d2, axis=1).astype(jnp.int32)[:, None]


def _vq_argmin(flat, cb, cb2t):
    n = flat.shape[0]
    return pl.pallas_call(
        _vq_body,
        out_shape=jax.ShapeDtypeStruct((n, 1), jnp.int32),
    )(flat, cb, cb2t)


def _cb2_body(cb_ref, out_ref):
    cb = cb_ref[...]
    cc = cb * cb
    out_ref[...] = jnp.transpose(_fold_sum_lanes(cc, 12))


def _cb2t(cb):
    return pl.pallas_call(
        _cb2_body,
        out_shape=jax.ShapeDtypeStruct((1, cb.shape[0]), jnp.float32),
    )(cb)


def _make_sc_gather(B, D):
    info = plsc.get_sparse_core_info()
    NC, NS = info.num_cores, info.num_subcores
    NW = NC * NS
    b_per_w = B // NW
    mesh = plsc.VectorSubcoreMesh(core_axis_name="c", subcore_axis_name="s")

    @functools.partial(
        pl.kernel, mesh=mesh,
        out_type=jax.ShapeDtypeStruct((B, D), jnp.float32),
        scratch_types=[
            pltpu.VMEM((b_per_w,), jnp.int32),
            pltpu.VMEM((b_per_w, D), jnp.float32),
            pltpu.SemaphoreType.DMA,
        ],
    )
    def k(table_hbm, idx_hbm, out_hbm, idx_v, rows_v, sem):
        wid = lax.axis_index("s") * NC + lax.axis_index("c")
        base = wid * b_per_w
        pltpu.sync_copy(idx_hbm.at[pl.ds(base, b_per_w)], idx_v)
        pltpu.async_copy(table_hbm.at[idx_v], rows_v, sem).wait()
        pltpu.sync_copy(rows_v, out_hbm.at[pl.ds(base, b_per_w)])

    return k


def kernel(x, feat_bias, feat_weight, emb_W1, emb_W2, emb_W3,
           attn_Wq, attn_Wk, attn_Wv, attn_Wo, ff_W1, ff_W2, out_W, codebook):
    vs = _featurize(x, feat_bias, feat_weight)
    vs = _lin(vs, emb_W1)
    vs = _act(vs)
    vs = _lin(vs, emb_W2)
    vs = _act(vs)
    vs = _lin(vs, emb_W3)
    vs = _pos_encoding(vs)
    for l in range(_NUM_LAYERS):
        vs = _encoder_layer(vs, attn_Wq[l], attn_Wk[l], attn_Wv[l],
                            attn_Wo[l], ff_W1[l], ff_W2[l])
    z_e = _fc_out(vs, out_W)

    B, S, D = z_e.shape
    flat = z_e.reshape(-1, D)
    cb2t = _cb2t(codebook)
    idx = _vq_argmin(flat, codebook, cb2t)[:, 0]

    cb_pad = jnp.concatenate(
        [codebook, jnp.zeros((codebook.shape[0], 116), jnp.float32)], axis=1)
    gathered = _make_sc_gather(B * S, 128)(cb_pad, idx)
    z_q = gathered[:, :D].reshape(B, S, D)

    z_q_st = z_e + lax.stop_gradient(z_q - z_e)
    return z_q_st, idx.reshape(B, S)


# speedup vs baseline: 1.1360x; 1.0622x over previous
import functools
import math

import jax
import jax.numpy as jnp
import numpy as np
from jax import lax
from jax.experimental import pallas as pl
from jax.experimental.pallas import tpu as pltpu
from jax.experimental.pallas import tpu_sc as plsc

_DIMS = [1, 1, 2, 2, 2, 2, 2]
_NUM_HEADS = 8
_NUM_LAYERS = 4
_MULT = 64
_MULT_FF = 256
_CODEBOOK = 8192


def _build_Q():
    j = np.arange(12)
    qs = [np.ones((1, 12)) / np.sqrt(12.0)]
    qs.append((((-1.0) ** j).reshape(1, 12)) / np.sqrt(12.0))
    for m in range(1, 6):
        c = np.cos(2.0 * np.pi * m * j / 12.0)
        s = np.sin(2.0 * np.pi * m * j / 12.0)
        qs.append(np.stack([c / np.linalg.norm(c), s / np.linalg.norm(s)], axis=0))
    return [jnp.asarray(q, dtype=jnp.float32) for q in qs]


_Q = _build_Q()


def _lin(vs, W):
    return [vs[i] @ W[i] for i in range(len(vs))]


def _act(vs):
    out = []
    for i in range(len(vs)):
        Q = _Q[i]
        p = jax.nn.gelu(jnp.einsum('db,...dm->...bm', Q, vs[i]), approximate=False)
        out.append(jnp.einsum('db,...bm->...dm', Q, p))
    return out


def _featurize(x, bias, weight):
    xv = (x + bias)[..., None]
    return [jnp.einsum('db,...bo->...do', Q, xv) * weight for Q in _Q]


def _pos_encoding(vs):
    S = vs[0].shape[1]
    position = jnp.arange(S, dtype=jnp.float32)[:, None]
    div = jnp.exp(jnp.arange(0, _MULT, 2, dtype=jnp.float32) * (-(math.log(10000.0) / _MULT)))
    pe = jnp.zeros((S, _MULT), dtype=jnp.float32)
    pe = pe.at[:, 0::2].set(jnp.sin(position * div))
    pe = pe.at[:, 1::2].set(jnp.cos(position * div))
    out = []
    for i in range(len(vs)):
        qsum = _Q[i].sum(axis=1)
        enc = qsum[None, :, None] * pe[:, None, :]
        out.append(vs[i] + enc[None])
    return out


_OFFS = [0, 1, 2, 4, 6, 8, 10]
_S = 512


def _attnA_body(*refs):
    v_refs = refs[0:7]
    wq_ref, wk_ref, wv_ref = refs[7:10]
    sc_ref = refs[10]
    vh_ref = refs[11]
    qs, ks, ps = [], [], []
    for i, d in enumerate(_DIMS):
        v_i = v_refs[i][0].reshape(_S * d, _MULT)
        q_i = jnp.dot(v_i, wq_ref[i], preferred_element_type=jnp.float32)
        k_i = jnp.dot(v_i, wk_ref[i], preferred_element_type=jnp.float32)
        p_i = jnp.dot(v_i, wv_ref[i], preferred_element_type=jnp.float32)
        qs.append(q_i.reshape(_S, d, _MULT))
        ks.append(k_i.reshape(_S, d, _MULT))
        ps.append(p_i.reshape(_S, d, _MULT))
    for h in range(_NUM_HEADS):
        qc = jnp.concatenate(
            [q[:, j, 8 * h:8 * h + 8] for q, d in zip(qs, _DIMS) for j in range(d)], axis=1)
        kc = jnp.concatenate(
            [k[:, j, 8 * h:8 * h + 8] for k, d in zip(ks, _DIMS) for j in range(d)], axis=1)
        vh_ref[0, h] = jnp.concatenate(
            [p[:, j, 8 * h:8 * h + 8] for p, d in zip(ps, _DIMS) for j in range(d)], axis=1)
        sc_ref[0, h] = lax.dot_general(
            qc, kc, (((1,), (1,)), ((), ())), preferred_element_type=jnp.float32)


def _attnA(vs, Wq, Wk, Wv):
    B = vs[0].shape[0]
    w_spec = pl.BlockSpec((7, _MULT, _MULT), lambda b: (0, 0, 0))
    v_specs = [pl.BlockSpec((1, _S, d, _MULT), lambda b: (b, 0, 0, 0)) for d in _DIMS]
    outs = pl.pallas_call(
        _attnA_body,
        grid=(B,),
        in_specs=v_specs + [w_spec, w_spec, w_spec],
        out_specs=[pl.BlockSpec((1, _NUM_HEADS, _S, _S), lambda b: (b, 0, 0, 0)),
                   pl.BlockSpec((1, _NUM_HEADS, _S, 96), lambda b: (b, 0, 0, 0))],
        out_shape=[jax.ShapeDtypeStruct((B, _NUM_HEADS, _S, _S), jnp.float32),
                   jax.ShapeDtypeStruct((B, _NUM_HEADS, _S, 96), jnp.float32)],
    )(*vs, Wq, Wk, Wv)
    return outs[0], outs[1]


def _split_heads(xs):
    hs = []
    for x in xs:
        B, S, d, m = x.shape
        mk = m // _NUM_HEADS
        hs.append(x.reshape(B, S, d, _NUM_HEADS, mk).transpose(0, 3, 1, 2, 4).reshape(B, _NUM_HEADS, S, d * mk))
    return hs


def _attnB_body(*refs):
    o_ref = refs[0]
    vs_refs = refs[1:8]
    wo_ref, f1_ref = refs[8], refs[9]
    f_refs = refs[10:17]
    v2_refs = refs[17:24]
    for i, d in enumerate(_DIMS):
        off = _OFFS[i]
        cols = []
        for h in range(_NUM_HEADS):
            cols.append(o_ref[0, h][:, 8 * off:8 * (off + d)])
        js = []
        for j in range(d):
            js.append(jnp.concatenate(
                [cols[h][:, 8 * j:8 * j + 8] for h in range(_NUM_HEADS)],
                axis=1)[:, None, :])
        attn_rows = jnp.concatenate(js, axis=1).reshape(_S * d, _MULT)
        a_i = jnp.dot(attn_rows, wo_ref[i], preferred_element_type=jnp.float32)
        v2_i = vs_refs[i][0].reshape(_S * d, _MULT) + a_i
        f_i = jnp.dot(v2_i, f1_ref[i], preferred_element_type=jnp.float32)
        v2_refs[i][0] = v2_i.reshape(_S, d, _MULT)
        f_refs[i][0] = f_i.reshape(_S, d, _MULT_FF)


def _attnB(o_cat, vs, Wo, F1):
    B = o_cat.shape[0]
    v_specs = [pl.BlockSpec((1, _S, d, _MULT), lambda b: (b, 0, 0, 0)) for d in _DIMS]
    f_specs = [pl.BlockSpec((1, _S, d, _MULT_FF), lambda b: (b, 0, 0, 0)) for d in _DIMS]
    outs = pl.pallas_call(
        _attnB_body,
        grid=(B,),
        in_specs=[pl.BlockSpec((1, _NUM_HEADS, _S, 96), lambda b: (b, 0, 0, 0))]
                 + v_specs
                 + [pl.BlockSpec((7, _MULT, _MULT), lambda b: (0, 0, 0)),
                    pl.BlockSpec((7, _MULT, _MULT_FF), lambda b: (0, 0, 0))],
        out_specs=f_specs + v_specs,
        out_shape=[jax.ShapeDtypeStruct((B, _S, d, _MULT_FF), jnp.float32) for d in _DIMS]
                  + [jax.ShapeDtypeStruct((B, _S, d, _MULT), jnp.float32) for d in _DIMS],
    )(o_cat, *vs, Wo, F1)
    return list(outs[:7]), list(outs[7:])


def _ffC_body(*refs):
    g_refs = refs[0:7]
    v2_refs = refs[7:14]
    f2_ref = refs[14]
    o_refs = refs[15:22]
    for i, d in enumerate(_DIMS):
        g_i = g_refs[i][0].reshape(_S * d, _MULT_FF)
        f_i = jnp.dot(g_i, f2_ref[i], preferred_element_type=jnp.float32)
        o_refs[i][0] = (v2_refs[i][0].reshape(_S * d, _MULT) + f_i).reshape(_S, d, _MULT)


def _ffC(g, v2, F2):
    B = g[0].shape[0]
    v_specs = [pl.BlockSpec((1, _S, d, _MULT), lambda b: (b, 0, 0, 0)) for d in _DIMS]
    f_specs = [pl.BlockSpec((1, _S, d, _MULT_FF), lambda b: (b, 0, 0, 0)) for d in _DIMS]
    outs = pl.pallas_call(
        _ffC_body,
        grid=(B,),
        in_specs=f_specs + v_specs
                 + [pl.BlockSpec((7, _MULT_FF, _MULT), lambda b: (0, 0, 0))],
        out_specs=v_specs,
        out_shape=[jax.ShapeDtypeStruct((B, _S, d, _MULT), jnp.float32) for d in _DIMS],
    )(*g, *v2, F2)
    return list(outs)


def _encoder_layer(vs, Wq, Wk, Wv, Wo, F1, F2):
    scores_raw, vh_cat = _attnA(vs, Wq, Wk, Wv)
    probs = jax.nn.softmax(scores_raw / math.sqrt(96), axis=-1)
    o_cat = jnp.matmul(probs, vh_cat)
    f, v2 = _attnB(o_cat, vs, Wo, F1)
    g = _act(f)
    return _ffC(g, v2, F2)


def _fc_out(vs, out_W):
    vs = _lin(vs, out_W)
    parts = [jnp.einsum('db,...dm->...bm', _Q[i], vs[i]) for i in range(len(vs))]
    perm = jnp.concatenate(parts, axis=-1)
    return jnp.mean(perm, axis=-1)


def _fold_sum_lanes(x, width):
    p = 1
    while p < width:
        p *= 2
    if p != width:
        x = jnp.concatenate(
            [x, jnp.zeros(x.shape[:-1] + (p - width,), x.dtype)], axis=-1)
    while p > 1:
        h = p // 2
        x = x[..., :h] + x[..., h:p]
        p = h
    return x


def _vq_body(flat_ref, cb_ref, cb2t_ref, idx_ref):
    flat = flat_ref[...]
    cb = cb_ref[...]
    ff = flat * flat
    a = _fold_sum_lanes(ff, 12)
    mm = lax.dot_general(flat, cb, (((1,), (1,)), ((), ())),
                         preferred_element_type=jnp.float32)
    d2 = (a - 2.0 * mm) + cb2t_ref[...]
    idx_ref[...] = jnp.argmin(d2, axis=1).astype(jnp.int32)[:, None]


def _vq_argmin(flat, cb, cb2t):
    n = flat.shape[0]
    return pl.pallas_call(
        _vq_body,
        out_shape=jax.ShapeDtypeStruct((n, 1), jnp.int32),
    )(flat, cb, cb2t)


def _cb2_body(cb_ref, out_ref):
    cb = cb_ref[...]
    cc = cb * cb
    out_ref[...] = jnp.transpose(_fold_sum_lanes(cc, 12))


def _cb2t(cb):
    return pl.pallas_call(
        _cb2_body,
        out_shape=jax.ShapeDtypeStruct((1, cb.shape[0]), jnp.float32),
    )(cb)


def _make_sc_gather(B, D):
    info = plsc.get_sparse_core_info()
    NC, NS = info.num_cores, info.num_subcores
    NW = NC * NS
    b_per_w = B // NW
    mesh = plsc.VectorSubcoreMesh(core_axis_name="c", subcore_axis_name="s")

    @functools.partial(
        pl.kernel, mesh=mesh,
        out_type=jax.ShapeDtypeStruct((B, D), jnp.float32),
        scratch_types=[
            pltpu.VMEM((b_per_w,), jnp.int32),
            pltpu.VMEM((b_per_w, D), jnp.float32),
            pltpu.SemaphoreType.DMA,
        ],
    )
    def k(table_hbm, idx_hbm, out_hbm, idx_v, rows_v, sem):
        wid = lax.axis_index("s") * NC + lax.axis_index("c")
        base = wid * b_per_w
        pltpu.sync_copy(idx_hbm.at[pl.ds(base, b_per_w)], idx_v)
        pltpu.async_copy(table_hbm.at[idx_v], rows_v, sem).wait()
        pltpu.sync_copy(rows_v, out_hbm.at[pl.ds(base, b_per_w)])

    return k


def kernel(x, feat_bias, feat_weight, emb_W1, emb_W2, emb_W3,
           attn_Wq, attn_Wk, attn_Wv, attn_Wo, ff_W1, ff_W2, out_W, codebook):
    vs = _featurize(x, feat_bias, feat_weight)
    vs = _lin(vs, emb_W1)
    vs = _act(vs)
    vs = _lin(vs, emb_W2)
    vs = _act(vs)
    vs = _lin(vs, emb_W3)
    vs = _pos_encoding(vs)
    for l in range(_NUM_LAYERS):
        vs = _encoder_layer(vs, attn_Wq[l], attn_Wk[l], attn_Wv[l],
                            attn_Wo[l], ff_W1[l], ff_W2[l])
    z_e = _fc_out(vs, out_W)

    B, S, D = z_e.shape
    flat = z_e.reshape(-1, D)
    cb2t = _cb2t(codebook)
    idx = _vq_argmin(flat, codebook, cb2t)[:, 0]

    cb_pad = jnp.concatenate(
        [codebook, jnp.zeros((codebook.shape[0], 116), jnp.float32)], axis=1)
    gathered = _make_sc_gather(B * S, 128)(cb_pad, idx)
    z_q = gathered[:, :D].reshape(B, S, D)

    z_q_st = z_e + lax.stop_gradient(z_q - z_e)
    return z_q_st, idx.reshape(B, S)
